# baseline (device time: 49650 ns/iter reference)
import jax
import jax.numpy as jnp
from jax import lax
from jax.experimental import pallas as pl
from jax.experimental.pallas import tpu as pltpu

N_DEV = 4


def _gemm(x, w):
    M, K = x.shape
    _, N = w.shape
    bk = 512

    def body(x_ref, w_ref, o_ref):
        @pl.when(pl.program_id(0) == 0)
        def _():
            o_ref[...] = jnp.zeros_like(o_ref)

        o_ref[...] += jnp.dot(
            x_ref[...], w_ref[...], preferred_element_type=jnp.float32
        )

    return pl.pallas_call(
        body,
        grid=(K // bk,),
        in_specs=[
            pl.BlockSpec((M, bk), lambda k: (0, k)),
            pl.BlockSpec((bk, N), lambda k: (k, 0)),
        ],
        out_specs=pl.BlockSpec((M, N), lambda k: (0, 0)),
        out_shape=jax.ShapeDtypeStruct((M, N), jnp.float32),
        compiler_params=pltpu.CompilerParams(
            dimension_semantics=("arbitrary",)
        ),
    )(x, w)


def _a2a_quant(y):
    m, n = y.shape
    nb = n // N_DEV

    def body(
        y_ref,
        o_ref,
        qsend_ref,
        qrecv_ref,
        amax_ref,
        qsend_sems,
        qrecv_sems,
        asend_sems,
        arecv_sems,
    ):
        my = lax.axis_index("i")

        amax = jnp.max(jnp.abs(y_ref[...]))
        amax_ref[0] = jnp.full((8, 128), amax, jnp.float32)

        barrier = pltpu.get_barrier_semaphore()
        for off in range(1, N_DEV):
            pl.semaphore_signal(
                barrier,
                inc=1,
                device_id=((my + off) % N_DEV,),
                device_id_type=pl.DeviceIdType.MESH,
            )
        pl.semaphore_wait(barrier, N_DEV - 1)

        amax_rdmas = []
        for off in range(1, N_DEV):
            rdma = pltpu.make_async_remote_copy(
                src_ref=amax_ref.at[0],
                dst_ref=amax_ref.at[off],
                send_sem=asend_sems.at[off],
                recv_sem=arecv_sems.at[off],
                device_id=((my + off) % N_DEV,),
                device_id_type=pl.DeviceIdType.MESH,
            )
            rdma.start()
            amax_rdmas.append(rdma)
        for rdma in amax_rdmas:
            rdma.wait()

        gmax = jnp.max(amax_ref[...])
        inv = 127.0 / gmax
        scale = gmax * (1.0 / 127.0)

        q = jnp.clip(jnp.round(y_ref[...] * inv), -127.0, 127.0)
        qsend_ref[...] = q.astype(jnp.int8)

        q_rdmas = []
        for off in range(1, N_DEV):
            dst = (my + off) % N_DEV
            rdma = pltpu.make_async_remote_copy(
                src_ref=qsend_ref.at[:, pl.ds(dst * nb, nb)],
                dst_ref=qrecv_ref.at[off],
                send_sem=qsend_sems.at[off],
                recv_sem=qrecv_sems.at[off],
                device_id=(dst,),
                device_id_type=pl.DeviceIdType.MESH,
            )
            rdma.start()
            q_rdmas.append(rdma)

        own = qsend_ref[:, pl.ds(my * nb, nb)]
        o_ref[pl.ds(my * m, m), :] = own.astype(jnp.float32) * scale

        for off in range(1, N_DEV):
            src = (my - off) % N_DEV
            q_rdmas[off - 1].wait()
            o_ref[pl.ds(src * m, m), :] = (
                qrecv_ref[off].astype(jnp.float32) * scale
            )

    return pl.pallas_call(
        body,
        out_shape=jax.ShapeDtypeStruct((N_DEV * m, nb), jnp.float32),
        in_specs=[pl.BlockSpec(memory_space=pltpu.VMEM)],
        out_specs=pl.BlockSpec(memory_space=pltpu.VMEM),
        scratch_shapes=[
            pltpu.VMEM((m, n), jnp.int8),
            pltpu.VMEM((N_DEV, m, nb), jnp.int8),
            pltpu.VMEM((N_DEV, 8, 128), jnp.float32),
            pltpu.SemaphoreType.DMA((N_DEV,)),
            pltpu.SemaphoreType.DMA((N_DEV,)),
            pltpu.SemaphoreType.DMA((N_DEV,)),
            pltpu.SemaphoreType.DMA((N_DEV,)),
        ],
        compiler_params=pltpu.CompilerParams(collective_id=0),
    )(y)


def kernel(x, w_mat):
    y = _gemm(x, w_mat)
    return _a2a_quant(y)


# device time: 46737 ns/iter; 1.0623x vs baseline; 1.0623x over previous
import jax
import jax.numpy as jnp
from jax import lax
from jax.experimental import pallas as pl
from jax.experimental.pallas import tpu as pltpu

N_DEV = 4


def kernel(x, w_mat):
    m, K = x.shape
    _, n = w_mat.shape
    nb = n // N_DEV

    offs_order = (2, 1, 3, 0)

    def body(
        x_ref,
        w_ref,
        o_ref,
        wbuf,
        sendbuf,
        recvbuf,
        amaxbuf,
        wsems,
        csend_sems,
        crecv_sems,
        asend_sems,
        arecv_sems,
    ):
        my = lax.axis_index("i")
        cols = [(my + off) % N_DEV for off in offs_order]

        def wcopy(step, slot):
            return pltpu.make_async_copy(
                w_ref.at[:, pl.ds(cols[step] * nb, nb)],
                wbuf.at[slot],
                wsems.at[slot],
            )

        wcopy(0, 0).start()
        wcopy(1, 1).start()

        barrier = pltpu.get_barrier_semaphore()
        for off in range(1, N_DEV):
            pl.semaphore_signal(
                barrier,
                inc=1,
                device_id=((my + off) % N_DEV,),
                device_id_type=pl.DeviceIdType.MESH,
            )
        pl.semaphore_wait(barrier, N_DEV - 1)

        chunk_rdmas = {}
        amax = None
        for step, off in enumerate(offs_order):
            slot = step % 2
            wcopy(step, slot).wait()
            yblk = jnp.dot(
                x_ref[...], wbuf[slot], preferred_element_type=jnp.float32
            )
            if step + 2 < N_DEV:
                wcopy(step + 2, slot).start()
            blk_amax = jnp.max(jnp.abs(yblk))
            amax = blk_amax if amax is None else jnp.maximum(amax, blk_amax)
            sendbuf[off] = yblk.astype(jnp.bfloat16)
            if off != 0:
                rdma = pltpu.make_async_remote_copy(
                    src_ref=sendbuf.at[off],
                    dst_ref=recvbuf.at[off - 1],
                    send_sem=csend_sems.at[off],
                    recv_sem=crecv_sems.at[off],
                    device_id=((my + off) % N_DEV,),
                    device_id_type=pl.DeviceIdType.MESH,
                )
                rdma.start()
                chunk_rdmas[off] = rdma

        amaxbuf[0] = jnp.full((8, 128), amax, jnp.float32)
        amax_rdmas = []
        for off in range(1, N_DEV):
            rdma = pltpu.make_async_remote_copy(
                src_ref=amaxbuf.at[0],
                dst_ref=amaxbuf.at[off],
                send_sem=asend_sems.at[off],
                recv_sem=arecv_sems.at[off],
                device_id=((my + off) % N_DEV,),
                device_id_type=pl.DeviceIdType.MESH,
            )
            rdma.start()
            amax_rdmas.append(rdma)
        for rdma in amax_rdmas:
            rdma.wait()

        gmax = jnp.max(amaxbuf[...])
        inv = 127.0 / gmax
        scale = gmax * (1.0 / 127.0)

        def store_rows(src_dev, blk_bf16):
            q = jnp.clip(jnp.round(blk_bf16.astype(jnp.float32) * inv),
                         -127.0, 127.0)
            o_ref[pl.ds(src_dev * m, m), :] = q * scale

        store_rows(my, sendbuf[0])

        for off in offs_order[:3]:
            chunk_rdmas[off].wait_recv()
            store_rows((my - off) % N_DEV, recvbuf[off - 1])
        for off in offs_order[:3]:
            chunk_rdmas[off].wait_send()

    return pl.pallas_call(
        body,
        out_shape=jax.ShapeDtypeStruct((N_DEV * m, nb), jnp.float32),
        in_specs=[
            pl.BlockSpec(memory_space=pltpu.VMEM),
            pl.BlockSpec(memory_space=pl.ANY),
        ],
        out_specs=pl.BlockSpec(memory_space=pltpu.VMEM),
        scratch_shapes=[
            pltpu.VMEM((2, K, nb), jnp.float32),
            pltpu.VMEM((N_DEV, m, nb), jnp.bfloat16),
            pltpu.VMEM((N_DEV - 1, m, nb), jnp.bfloat16),
            pltpu.VMEM((N_DEV, 8, 128), jnp.float32),
            pltpu.SemaphoreType.DMA((2,)),
            pltpu.SemaphoreType.DMA((N_DEV,)),
            pltpu.SemaphoreType.DMA((N_DEV,)),
            pltpu.SemaphoreType.DMA((N_DEV,)),
            pltpu.SemaphoreType.DMA((N_DEV,)),
        ],
        compiler_params=pltpu.CompilerParams(
            collective_id=0,
            vmem_limit_bytes=44 * 1024 * 1024,
        ),
    )(x, w_mat)
